# baseline (speedup 1.0000x reference)
import functools

import jax
import jax.numpy as jnp
from jax import lax
from jax.experimental import pallas as pl
from jax.experimental.pallas import tpu as pltpu
from jax.experimental.pallas import tpu_sc as plsc

_N = 4096
_NW = 32
_ROWS = _N // _NW
_CR = 8
_NCH = _ROWS // _CR
_NJ = _N // 16

_BQ = 512


def _sc_a_pass(A_hbm, x_hbm, y_hbm, ax_hbm, atyp_hbm,
               x_v, aty_v, y_v, ax_v, abuf, sem0, sem1):
    wid = lax.axis_index("s") * 2 + lax.axis_index("c")
    row0 = wid * _ROWS
    pltpu.sync_copy(x_hbm, x_v)
    pltpu.sync_copy(y_hbm.at[pl.ds(row0, _ROWS)], y_v)

    def _zero(j, _):
        aty_v[pl.ds(j * 16, 16)] = jnp.zeros((16,), jnp.float32)
        return 0

    lax.fori_loop(0, _NJ, _zero, 0)

    sems = (sem0, sem1)
    iota16 = lax.iota(jnp.int32, 16)
    pend = jnp.zeros((16,), jnp.float32)
    dmas = [None, None]
    dmas[0] = pltpu.make_async_copy(
        A_hbm.at[pl.ds(row0, _CR)], abuf.at[0], sem0)
    dmas[0].start()
    for c in range(_NCH):
        cur = c % 2
        nxt = (c + 1) % 2
        if c + 1 < _NCH:
            dmas[nxt] = pltpu.make_async_copy(
                A_hbm.at[pl.ds(row0 + (c + 1) * _CR, _CR)],
                abuf.at[nxt], sems[nxt])
            dmas[nxt].start()
        dmas[cur].wait()
        buf = abuf.at[cur]
        yv16 = y_v[pl.ds(((c * _CR) // 16) * 16, 16)]
        ybc = [jnp.broadcast_to(yv16[(c * _CR + r) % 16], (16,))
               for r in range(_CR)]

        def _col(j, dots):
            sl = pl.ds(j * 16, 16)
            xv = x_v[sl]
            atyv = aty_v[sl]
            new_dots = []
            for r in range(_CR):
                av = buf[r, sl]
                new_dots.append(dots[r] + av * xv)
                atyv = atyv + ybc[r] * av
            aty_v[sl] = atyv
            return tuple(new_dots)

        zero16 = jnp.zeros((16,), jnp.float32)
        dots = lax.fori_loop(0, _NJ, _col, (zero16,) * _CR)
        for r in range(_CR):
            s = jnp.sum(dots[r], axis=0)
            lane = (c % 2) * _CR + r
            pend = jnp.where(iota16 == lane, jnp.broadcast_to(s, (16,)),
                             pend)
        if c % 2 == 1:
            ax_v[pl.ds((c - 1) * _CR, 16)] = pend

    pltpu.sync_copy(ax_v, ax_hbm.at[pl.ds(row0, _ROWS)])
    pltpu.sync_copy(aty_v, atyp_hbm.at[wid])


def _qx_body(xT_ref, Q_ref, Qx_ref):
    xT = xT_ref[...]
    Q = Q_ref[...]
    Qx_ref[...] = jax.lax.dot_general(xT, Q, (((1,), (1,)), ((), ())),
                                      preferred_element_type=jnp.float32)


def _finalize_body(Ax_ref, Qx_ref, ATyp_ref, b_ref, c_ref, x_ref, y_ref,
                   Iy_ref, il_ref, iu_ref, l_ref, u_ref,
                   res_ref, t1_ref, t2_ref, t3_ref):
    relu = lambda v: jnp.maximum(v, 0.0)
    Ax = Ax_ref[...]
    Qx = Qx_ref[...]
    ATy = jnp.sum(ATyp_ref[...], axis=0)
    b = b_ref[...]
    c = c_ref[...]
    x = x_ref[...]
    y = y_ref[...]
    Iy = Iy_ref[...]
    il = il_ref[...]
    iu = iu_ref[...]
    l = l_ref[...]
    u = u_ref[...]
    cons = Ax - b
    cons = cons + relu(-cons) * Iy
    var = relu(l - x) * il + relu(x - u) * iu
    part2 = jnp.maximum(jnp.max(jnp.abs(var)), jnp.max(jnp.abs(cons)))
    t1 = part2 / (1.0 + jnp.max(jnp.abs(b)))
    quad = jnp.sum(x * Qx)
    lin = jnp.sum(c * x)
    vio = jnp.sum(b * y)
    pg_g = c - ATy + Qx
    RC = relu(pg_g) * il - relu(-pg_g) * iu
    tm = jnp.where(RC > 0, l, u)
    rc = jnp.sum(RC * tm)
    top_g = jnp.abs(quad + lin - vio - rc)
    bot_g = 1.0 + jnp.maximum(jnp.abs(vio - 0.5 * quad),
                              jnp.abs(0.5 * quad + lin))
    t3 = top_g / bot_g
    pg = c + ATy + Qx
    RCV = pg - relu(pg) * il - relu(-pg) * iu
    DR = relu(-y) * Iy
    t2 = jnp.maximum(jnp.max(jnp.abs(RCV)), jnp.max(jnp.abs(DR))) / \
        (1.0 + jnp.max(jnp.abs(c)))
    res_ref[...] = jnp.reshape(t1 + t2 + t3, (1, 1))
    t1_ref[...] = jnp.reshape(t1, (1, 1))
    t2_ref[...] = jnp.reshape(t2, (1, 1))
    t3_ref[...] = jnp.reshape(t3, (1, 1))


def kernel(Q, A, AT, b, c, x, y, Iy, il, iu, l, u):
    del AT
    m, n = A.shape

    sc_call = pl.kernel(
        _sc_a_pass,
        out_type=[
            jax.ShapeDtypeStruct((m,), jnp.float32),
            jax.ShapeDtypeStruct((_NW, n), jnp.float32),
        ],
        mesh=plsc.VectorSubcoreMesh(core_axis_name="c", subcore_axis_name="s"),
        scratch_types=[
            pltpu.VMEM((n,), jnp.float32),
            pltpu.VMEM((n,), jnp.float32),
            pltpu.VMEM((_ROWS,), jnp.float32),
            pltpu.VMEM((_ROWS,), jnp.float32),
            pltpu.VMEM((2, _CR, n), jnp.float32),
            pltpu.SemaphoreType.DMA,
            pltpu.SemaphoreType.DMA,
        ],
        compiler_params=pltpu.CompilerParams(needs_layout_passes=False),
    )
    Ax, ATyp = sc_call(A, x.reshape(m), y.reshape(m))

    xT = x.reshape(1, n)
    Qx = pl.pallas_call(
        _qx_body,
        grid=(m // _BQ,),
        in_specs=[
            pl.BlockSpec((1, n), lambda i: (0, 0)),
            pl.BlockSpec((_BQ, n), lambda i: (i, 0)),
        ],
        out_specs=pl.BlockSpec((1, _BQ), lambda i: (0, i)),
        out_shape=jax.ShapeDtypeStruct((1, n), jnp.float32),
    )(xT, Q)

    shp = (32, n // 32)
    sd = jax.ShapeDtypeStruct((1, 1), jnp.float32)
    res, t1, t2, t3 = pl.pallas_call(
        _finalize_body,
        out_shape=[sd, sd, sd, sd],
    )(Ax.reshape(shp), Qx.reshape(shp), ATyp.reshape(_NW, 32, n // 32),
      b.reshape(shp), c.reshape(shp), x.reshape(shp), y.reshape(shp),
      Iy.reshape(shp), il.reshape(shp), iu.reshape(shp),
      l.reshape(shp), u.reshape(shp))
    return (res, t1.reshape(()), t2.reshape(()), t3)


# speedup vs baseline: 1.4282x; 1.4282x over previous
import jax
import jax.numpy as jnp
from jax import lax
from jax.experimental import pallas as pl
from jax.experimental.pallas import tpu as pltpu
from jax.experimental.pallas import tpu_sc as plsc

_N = 4096
_NW = 32
_RSC = 2048
_RW = _RSC // _NW
_CR = 8
_NCH = _RW // _CR
_NJ = _N // 16

_BA = 256
_BQ = 512


def _sc_q_pass(Q_hbm, x_hbm, qx_hbm, x_v, qx_v, qbuf, sem0, sem1):
    wid = lax.axis_index("s") * 2 + lax.axis_index("c")
    row0 = wid * _RW
    pltpu.sync_copy(x_hbm, x_v)

    sems = (sem0, sem1)
    iota16 = lax.iota(jnp.int32, 16)
    pend = jnp.zeros((16,), jnp.float32)
    dmas = [None, None]
    dmas[0] = pltpu.make_async_copy(
        Q_hbm.at[pl.ds(row0, _CR)], qbuf.at[0], sem0)
    dmas[0].start()
    for c in range(_NCH):
        cur = c % 2
        nxt = (c + 1) % 2
        if c + 1 < _NCH:
            dmas[nxt] = pltpu.make_async_copy(
                Q_hbm.at[pl.ds(row0 + (c + 1) * _CR, _CR)],
                qbuf.at[nxt], sems[nxt])
            dmas[nxt].start()
        dmas[cur].wait()
        buf = qbuf.at[cur]

        def _col(j, dots):
            new_dots = list(dots)
            for h in range(2):
                sl = pl.ds(j * 32 + h * 16, 16)
                xv = x_v[sl]
                for r in range(_CR):
                    new_dots[r] = new_dots[r] + buf[r, sl] * xv
            return tuple(new_dots)

        zero16 = jnp.zeros((16,), jnp.float32)
        dots = lax.fori_loop(0, _NJ // 2, _col, (zero16,) * _CR)
        for r in range(_CR):
            s = jnp.sum(dots[r], axis=0)
            lane = (c % 2) * _CR + r
            pend = jnp.where(iota16 == lane, jnp.broadcast_to(s, (16,)),
                             pend)
        if c % 2 == 1:
            qx_v[pl.ds((c - 1) * _CR, 16)] = pend

    pltpu.sync_copy(qx_v, qx_hbm.at[pl.ds(row0, _RW)])


def _a_body(xT_ref, A_ref, yblk_ref, Ax_ref, ATy_ref):
    i = pl.program_id(0)
    A = A_ref[...]
    xT = xT_ref[...]
    yb = yblk_ref[...]
    Ax_ref[...] = jax.lax.dot_general(xT, A, (((1,), (1,)), ((), ())),
                                      preferred_element_type=jnp.float32)
    contrib = jax.lax.dot_general(yb, A, (((1,), (0,)), ((), ())),
                                  preferred_element_type=jnp.float32)

    @pl.when(i == 0)
    def _init():
        ATy_ref[...] = contrib

    @pl.when(i > 0)
    def _acc():
        ATy_ref[...] = ATy_ref[...] + contrib


def _qx_body(xT_ref, Q_ref, Qx_ref):
    xT = xT_ref[...]
    Q = Q_ref[...]
    Qx_ref[...] = jax.lax.dot_general(xT, Q, (((1,), (1,)), ((), ())),
                                      preferred_element_type=jnp.float32)


def _finalize_body(Ax_ref, Qx_ref, ATy_ref, b_ref, c_ref, x_ref, y_ref,
                   Iy_ref, il_ref, iu_ref, l_ref, u_ref,
                   res_ref, t1_ref, t2_ref, t3_ref):
    relu = lambda v: jnp.maximum(v, 0.0)
    Ax = Ax_ref[...]
    Qx = Qx_ref[...]
    ATy = ATy_ref[...]
    b = b_ref[...]
    c = c_ref[...]
    x = x_ref[...]
    y = y_ref[...]
    Iy = Iy_ref[...]
    il = il_ref[...]
    iu = iu_ref[...]
    l = l_ref[...]
    u = u_ref[...]
    cons = Ax - b
    cons = cons + relu(-cons) * Iy
    var = relu(l - x) * il + relu(x - u) * iu
    part2 = jnp.maximum(jnp.max(jnp.abs(var)), jnp.max(jnp.abs(cons)))
    t1 = part2 / (1.0 + jnp.max(jnp.abs(b)))
    quad = jnp.sum(x * Qx)
    lin = jnp.sum(c * x)
    vio = jnp.sum(b * y)
    pg_g = c - ATy + Qx
    RC = relu(pg_g) * il - relu(-pg_g) * iu
    tm = jnp.where(RC > 0, l, u)
    rc = jnp.sum(RC * tm)
    top_g = jnp.abs(quad + lin - vio - rc)
    bot_g = 1.0 + jnp.maximum(jnp.abs(vio - 0.5 * quad),
                              jnp.abs(0.5 * quad + lin))
    t3 = top_g / bot_g
    pg = c + ATy + Qx
    RCV = pg - relu(pg) * il - relu(-pg) * iu
    DR = relu(-y) * Iy
    t2 = jnp.maximum(jnp.max(jnp.abs(RCV)), jnp.max(jnp.abs(DR))) / \
        (1.0 + jnp.max(jnp.abs(c)))
    res_ref[...] = jnp.reshape(t1 + t2 + t3, (1, 1))
    t1_ref[...] = jnp.reshape(t1, (1, 1))
    t2_ref[...] = jnp.reshape(t2, (1, 1))
    t3_ref[...] = jnp.reshape(t3, (1, 1))


def kernel(Q, A, AT, b, c, x, y, Iy, il, iu, l, u):
    del AT
    m, n = A.shape

    sc_call = pl.kernel(
        _sc_q_pass,
        out_type=jax.ShapeDtypeStruct((_RSC,), jnp.float32),
        mesh=plsc.VectorSubcoreMesh(core_axis_name="c", subcore_axis_name="s"),
        scratch_types=[
            pltpu.VMEM((n,), jnp.float32),
            pltpu.VMEM((_RW,), jnp.float32),
            pltpu.VMEM((2, _CR, n), jnp.float32),
            pltpu.SemaphoreType.DMA,
            pltpu.SemaphoreType.DMA,
        ],
        compiler_params=pltpu.CompilerParams(needs_layout_passes=False),
    )
    Qx_sc = sc_call(Q, x.reshape(m))

    xT = x.reshape(1, n)
    yT = y.reshape(1, m)
    Ax, ATy = pl.pallas_call(
        _a_body,
        grid=(m // _BA,),
        in_specs=[
            pl.BlockSpec((1, n), lambda i: (0, 0)),
            pl.BlockSpec((_BA, n), lambda i: (i, 0)),
            pl.BlockSpec((1, _BA), lambda i: (0, i)),
        ],
        out_specs=[
            pl.BlockSpec((1, _BA), lambda i: (0, i)),
            pl.BlockSpec((1, n), lambda i: (0, 0)),
        ],
        out_shape=[
            jax.ShapeDtypeStruct((1, m), jnp.float32),
            jax.ShapeDtypeStruct((1, n), jnp.float32),
        ],
    )(xT, A, yT)

    nqr = m - _RSC
    Qx_tc = pl.pallas_call(
        _qx_body,
        grid=(nqr // _BQ,),
        in_specs=[
            pl.BlockSpec((1, n), lambda i: (0, 0)),
            pl.BlockSpec((_BQ, n), lambda i: (i + _RSC // _BQ, 0)),
        ],
        out_specs=pl.BlockSpec((1, _BQ), lambda i: (0, i)),
        out_shape=jax.ShapeDtypeStruct((1, nqr), jnp.float32),
    )(xT, Q)

    Qx = jnp.concatenate([Qx_sc, Qx_tc.reshape(nqr)])

    shp = (32, n // 32)
    sd = jax.ShapeDtypeStruct((1, 1), jnp.float32)
    res, t1, t2, t3 = pl.pallas_call(
        _finalize_body,
        out_shape=[sd, sd, sd, sd],
    )(Ax.reshape(shp), Qx.reshape(shp), ATy.reshape(shp),
      b.reshape(shp), c.reshape(shp), x.reshape(shp), y.reshape(shp),
      Iy.reshape(shp), il.reshape(shp), iu.reshape(shp),
      l.reshape(shp), u.reshape(shp))
    return (res, t1.reshape(()), t2.reshape(()), t3)


# speedup vs baseline: 2.1038x; 1.4731x over previous
import jax
import jax.numpy as jnp
from jax.experimental import pallas as pl

_B = 256
_H = 2048


def _matvec_body(xT_ref, A0_ref, A1_ref, Q0_ref, Q1_ref, yblk_ref,
                 Ax_ref, Qx_ref, ATy_ref):
    i = pl.program_id(0)
    xT = xT_ref[...]
    yb = yblk_ref[...]
    x0 = xT[:, :_H]
    x1 = xT[:, _H:]
    dn_row = (((1,), (1,)), ((), ()))
    dn_col = (((1,), (0,)), ((), ()))
    A0 = A0_ref[...]
    A1 = A1_ref[...]
    Q0 = Q0_ref[...]
    Q1 = Q1_ref[...]
    f32 = jnp.float32
    Ax_ref[...] = (
        jax.lax.dot_general(x0, A0, dn_row, preferred_element_type=f32)
        + jax.lax.dot_general(x1, A1, dn_row, preferred_element_type=f32))
    Qx_ref[...] = (
        jax.lax.dot_general(x0, Q0, dn_row, preferred_element_type=f32)
        + jax.lax.dot_general(x1, Q1, dn_row, preferred_element_type=f32))
    c0 = jax.lax.dot_general(yb, A0, dn_col, preferred_element_type=f32)
    c1 = jax.lax.dot_general(yb, A1, dn_col, preferred_element_type=f32)

    @pl.when(i == 0)
    def _init():
        ATy_ref[:, :_H] = c0
        ATy_ref[:, _H:] = c1

    @pl.when(i > 0)
    def _acc():
        ATy_ref[:, :_H] = ATy_ref[:, :_H] + c0
        ATy_ref[:, _H:] = ATy_ref[:, _H:] + c1


def _finalize_body(Ax_ref, Qx_ref, ATy_ref, b_ref, c_ref, x_ref, y_ref,
                   Iy_ref, il_ref, iu_ref, l_ref, u_ref,
                   res_ref, t1_ref, t2_ref, t3_ref):
    relu = lambda v: jnp.maximum(v, 0.0)
    Ax = Ax_ref[...]
    Qx = Qx_ref[...]
    ATy = ATy_ref[...]
    b = b_ref[...]
    c = c_ref[...]
    x = x_ref[...]
    y = y_ref[...]
    Iy = Iy_ref[...]
    il = il_ref[...]
    iu = iu_ref[...]
    l = l_ref[...]
    u = u_ref[...]
    cons = Ax - b
    cons = cons + relu(-cons) * Iy
    var = relu(l - x) * il + relu(x - u) * iu
    part2 = jnp.maximum(jnp.max(jnp.abs(var)), jnp.max(jnp.abs(cons)))
    t1 = part2 / (1.0 + jnp.max(jnp.abs(b)))
    quad = jnp.sum(x * Qx)
    lin = jnp.sum(c * x)
    vio = jnp.sum(b * y)
    pg_g = c - ATy + Qx
    RC = relu(pg_g) * il - relu(-pg_g) * iu
    tm = jnp.where(RC > 0, l, u)
    rc = jnp.sum(RC * tm)
    top_g = jnp.abs(quad + lin - vio - rc)
    bot_g = 1.0 + jnp.maximum(jnp.abs(vio - 0.5 * quad),
                              jnp.abs(0.5 * quad + lin))
    t3 = top_g / bot_g
    pg = c + ATy + Qx
    RCV = pg - relu(pg) * il - relu(-pg) * iu
    DR = relu(-y) * Iy
    t2 = jnp.maximum(jnp.max(jnp.abs(RCV)), jnp.max(jnp.abs(DR))) / \
        (1.0 + jnp.max(jnp.abs(c)))
    res_ref[...] = jnp.reshape(t1 + t2 + t3, (1, 1))
    t1_ref[...] = jnp.reshape(t1, (1, 1))
    t2_ref[...] = jnp.reshape(t2, (1, 1))
    t3_ref[...] = jnp.reshape(t3, (1, 1))


def kernel(Q, A, AT, b, c, x, y, Iy, il, iu, l, u):
    del AT
    m, n = A.shape
    nb = m // _B
    xT = x.reshape(1, n)
    yT = y.reshape(1, m)

    Ax, Qx, ATy = pl.pallas_call(
        _matvec_body,
        grid=(nb,),
        in_specs=[
            pl.BlockSpec((1, n), lambda i: (0, 0)),
            pl.BlockSpec((_B, _H), lambda i: (i, 0)),
            pl.BlockSpec((_B, _H), lambda i: (i, 1)),
            pl.BlockSpec((_B, _H), lambda i: (i, 0)),
            pl.BlockSpec((_B, _H), lambda i: (i, 1)),
            pl.BlockSpec((1, _B), lambda i: (0, i)),
        ],
        out_specs=[
            pl.BlockSpec((1, _B), lambda i: (0, i)),
            pl.BlockSpec((1, _B), lambda i: (0, i)),
            pl.BlockSpec((1, n), lambda i: (0, 0)),
        ],
        out_shape=[
            jax.ShapeDtypeStruct((1, m), jnp.float32),
            jax.ShapeDtypeStruct((1, n), jnp.float32),
            jax.ShapeDtypeStruct((1, n), jnp.float32),
        ],
    )(xT, A, A, Q, Q, yT)

    shp = (32, n // 32)
    sd = jax.ShapeDtypeStruct((1, 1), jnp.float32)
    res, t1, t2, t3 = pl.pallas_call(
        _finalize_body,
        out_shape=[sd, sd, sd, sd],
    )(Ax.reshape(shp), Qx.reshape(shp), ATy.reshape(shp),
      b.reshape(shp), c.reshape(shp), x.reshape(shp), y.reshape(shp),
      Iy.reshape(shp), il.reshape(shp), iu.reshape(shp),
      l.reshape(shp), u.reshape(shp))
    return (res, t1.reshape(()), t2.reshape(()), t3)
